# Initial kernel scaffold; baseline (speedup 1.0000x reference)
import jax
import jax.numpy as jnp
from jax.experimental import pallas as pl


def kernel(adj_p, x, w1, b1, w2, b2, wfc, bfc):
    raise NotImplementedError("write your pallas kernel here")


# speedup vs baseline: 1.0693x; 1.0693x over previous
import functools

import jax
import jax.numpy as jnp
from jax.experimental import pallas as pl
from jax.experimental.pallas import tpu as pltpu

_LANE = 128
_VMEM_LIMIT = 48 * 1024 * 1024
_TM = 512
_TK = 2048
_TZ = 1024


def _round_up(n, m):
    return ((n + m - 1) // m) * m


def _tile(n, cap):
    t = min(cap, n)
    while t > _LANE:
        if n % t == 0:
            return t
        t -= _LANE
    return _LANE


def _pad2(a, rows, cols):
    r, c = a.shape
    if (r, c) == (rows, cols):
        return a
    return jnp.pad(a, ((0, rows - r), (0, cols - c)))


def _z1_kernel(x_ref, w1_ref, z1_ref):
    xb = x_ref[...].astype(jnp.bfloat16)
    z1_ref[...] = jnp.dot(xb, w1_ref[...],
                          preferred_element_type=jnp.float32).astype(jnp.bfloat16)


def _agg_kernel(adj_ref, z1_ref, b1_ref, w2_ref, b2_ref, wfc_ref, bfc_ref,
                out_ref, z2_ref, acc_ref, *, tm, tk, nk):
    p = pl.program_id(0)
    i = pl.program_id(1)
    k = pl.program_id(2)

    @pl.when(k == 0)
    def _():
        acc_ref[...] = jnp.zeros_like(acc_ref)

    @pl.when(p == 0)
    def _():
        zk = z1_ref[pl.ds(pl.multiple_of(k * tk, _LANE), tk), :]
        acc_ref[...] += jnp.dot(adj_ref[...], zk,
                                preferred_element_type=jnp.float32)

    @pl.when(p == 1)
    def _():
        zk = z2_ref[pl.ds(pl.multiple_of(k * tk, _LANE), tk), :]
        acc_ref[...] += jnp.dot(adj_ref[...], zk,
                                preferred_element_type=jnp.float32)

    @pl.when((k == nk - 1) & (p == 0))
    def _():
        h = jnp.maximum(acc_ref[...] + b1_ref[...], 0.0)
        z2_ref[pl.ds(pl.multiple_of(i * tm, _LANE), tm), :] = jnp.dot(
            h.astype(jnp.bfloat16), w2_ref[...],
            preferred_element_type=jnp.float32).astype(jnp.bfloat16)

    @pl.when((k == nk - 1) & (p == 1))
    def _():
        h = acc_ref[...] + b2_ref[...]
        out_ref[...] = jnp.dot(h.astype(jnp.bfloat16), wfc_ref[...],
                               preferred_element_type=jnp.float32) + bfc_ref[...]


@jax.jit
def _forward(adj_p, x, w1, b1, w2, b2, wfc, bfc):
    n, nfeat = x.shape
    npad = adj_p.shape[0]
    nhid = w1.shape[1]
    nclass = wfc.shape[1]

    fpad = _round_up(nfeat, _LANE)
    hpad = _round_up(nhid, _LANE)
    cpad = _round_up(nclass, _LANE)

    x_p = _pad2(x, npad, fpad)
    w1_p = _pad2(w1, fpad, hpad).astype(jnp.bfloat16)
    b1_p = _pad2(b1, 1, hpad).astype(jnp.float32)
    w2_p = _pad2(w2, hpad, hpad).astype(jnp.bfloat16)
    b2_p = _pad2(b2, 1, hpad).astype(jnp.float32)
    wfc_p = _pad2(wfc, hpad, cpad).astype(jnp.bfloat16)
    bfc_p = _pad2(bfc, 1, cpad).astype(jnp.float32)

    tz = _tile(npad, _TZ)
    tm = _tile(npad, _TM)
    tk = _tile(npad, _TK)
    nk = npad // tk

    z1 = pl.pallas_call(
        _z1_kernel,
        out_shape=jax.ShapeDtypeStruct((npad, hpad), jnp.bfloat16),
        grid=(npad // tz,),
        in_specs=[
            pl.BlockSpec((tz, fpad), lambda i: (i, 0)),
            pl.BlockSpec((fpad, hpad), lambda i: (0, 0)),
        ],
        out_specs=pl.BlockSpec((tz, hpad), lambda i: (i, 0)),
        compiler_params=pltpu.CompilerParams(
            dimension_semantics=("arbitrary",),
            vmem_limit_bytes=_VMEM_LIMIT,
        ),
    )(x_p, w1_p)

    body = functools.partial(_agg_kernel, tm=tm, tk=tk, nk=nk)
    out = pl.pallas_call(
        body,
        out_shape=jax.ShapeDtypeStruct((npad, cpad), jnp.float32),
        grid=(2, npad // tm, nk),
        in_specs=[
            pl.BlockSpec((tm, tk), lambda p, i, k: (i, k)),
            pl.BlockSpec((npad, hpad), lambda p, i, k: (0, 0)),
            pl.BlockSpec((1, hpad), lambda p, i, k: (0, 0)),
            pl.BlockSpec((hpad, hpad), lambda p, i, k: (0, 0)),
            pl.BlockSpec((1, hpad), lambda p, i, k: (0, 0)),
            pl.BlockSpec((hpad, cpad), lambda p, i, k: (0, 0)),
            pl.BlockSpec((1, cpad), lambda p, i, k: (0, 0)),
        ],
        out_specs=pl.BlockSpec((tm, cpad), lambda p, i, k: (p * i, 0)),
        scratch_shapes=[
            pltpu.VMEM((npad, hpad), jnp.bfloat16),
            pltpu.VMEM((tm, hpad), jnp.float32),
        ],
        compiler_params=pltpu.CompilerParams(
            dimension_semantics=("arbitrary", "arbitrary", "arbitrary"),
            vmem_limit_bytes=_VMEM_LIMIT,
        ),
    )(adj_p, z1, b1_p, w2_p, b2_p, wfc_p, bfc_p)

    return out[:n, :nclass]


def kernel(adj_p, x, w1, b1, w2, b2, wfc, bfc):
    return _forward(adj_p, x, w1, b1, w2, b2, wfc, bfc)


# speedup vs baseline: 1.6910x; 1.5813x over previous
import functools

import jax
import jax.numpy as jnp
from jax.experimental import pallas as pl
from jax.experimental.pallas import tpu as pltpu

_LANE = 128
_VMEM_LIMIT = 48 * 1024 * 1024
_TM = 512
_TK = 2048
_TZ = 1024


def _round_up(n, m):
    return ((n + m - 1) // m) * m


def _tile(n, cap):
    t = min(cap, n)
    while t > _LANE:
        if n % t == 0:
            return t
        t -= _LANE
    return _LANE


def _pad2(a, rows, cols):
    r, c = a.shape
    if (r, c) == (rows, cols):
        return a
    return jnp.pad(a, ((0, rows - r), (0, cols - c)))


def _z1_kernel(x_ref, w1_ref, z1_ref):
    xb = x_ref[...].astype(jnp.bfloat16)
    z1_ref[...] = jnp.dot(xb, w1_ref[...],
                          preferred_element_type=jnp.float32).astype(jnp.bfloat16)


def _agg_kernel(adj_ref, z1_ref, b1_ref, w2_ref, b2_ref, wfc_ref, bfc_ref,
                out_ref, z2_ref, *, tm):
    p = pl.program_id(0)
    i = pl.program_id(1)

    @pl.when(p == 0)
    def _():
        acc = jnp.dot(adj_ref[...], z1_ref[...],
                      preferred_element_type=jnp.float32)
        h = jnp.maximum(acc + b1_ref[...], 0.0)
        z2_ref[pl.ds(pl.multiple_of(i * tm, _LANE), tm), :] = jnp.dot(
            h.astype(jnp.bfloat16), w2_ref[...],
            preferred_element_type=jnp.float32).astype(jnp.bfloat16)

    @pl.when(p == 1)
    def _():
        acc = jnp.dot(adj_ref[...], z2_ref[...],
                      preferred_element_type=jnp.float32)
        h = acc + b2_ref[...]
        out_ref[...] = jnp.dot(h.astype(jnp.bfloat16), wfc_ref[...],
                               preferred_element_type=jnp.float32) + bfc_ref[...]


@jax.jit
def _forward(adj_p, x, w1, b1, w2, b2, wfc, bfc):
    n, nfeat = x.shape
    npad = adj_p.shape[0]
    nhid = w1.shape[1]
    nclass = wfc.shape[1]

    fpad = _round_up(nfeat, _LANE)
    hpad = _round_up(nhid, _LANE)
    cpad = _round_up(nclass, _LANE)

    x_p = _pad2(x, npad, fpad)
    w1_p = _pad2(w1, fpad, hpad).astype(jnp.bfloat16)
    b1_p = _pad2(b1, 1, hpad).astype(jnp.float32)
    w2_p = _pad2(w2, hpad, hpad).astype(jnp.bfloat16)
    b2_p = _pad2(b2, 1, hpad).astype(jnp.float32)
    wfc_p = _pad2(wfc, hpad, cpad).astype(jnp.bfloat16)
    bfc_p = _pad2(bfc, 1, cpad).astype(jnp.float32)

    tz = _tile(npad, _TZ)
    tm = _tile(npad, _TM)

    z1 = pl.pallas_call(
        _z1_kernel,
        out_shape=jax.ShapeDtypeStruct((npad, hpad), jnp.bfloat16),
        grid=(npad // tz,),
        in_specs=[
            pl.BlockSpec((tz, fpad), lambda i: (i, 0)),
            pl.BlockSpec((fpad, hpad), lambda i: (0, 0)),
        ],
        out_specs=pl.BlockSpec((tz, hpad), lambda i: (i, 0)),
        compiler_params=pltpu.CompilerParams(
            dimension_semantics=("arbitrary",),
            vmem_limit_bytes=_VMEM_LIMIT,
        ),
    )(x_p, w1_p)

    body = functools.partial(_agg_kernel, tm=tm)
    out = pl.pallas_call(
        body,
        out_shape=jax.ShapeDtypeStruct((npad, cpad), jnp.float32),
        grid=(2, npad // tm),
        in_specs=[
            pl.BlockSpec((tm, npad), lambda p, i: (i, 0)),
            pl.BlockSpec((npad, hpad), lambda p, i: (0, 0)),
            pl.BlockSpec((1, hpad), lambda p, i: (0, 0)),
            pl.BlockSpec((hpad, hpad), lambda p, i: (0, 0)),
            pl.BlockSpec((1, hpad), lambda p, i: (0, 0)),
            pl.BlockSpec((hpad, cpad), lambda p, i: (0, 0)),
            pl.BlockSpec((1, cpad), lambda p, i: (0, 0)),
        ],
        out_specs=pl.BlockSpec((tm, cpad), lambda p, i: (p * i, 0)),
        scratch_shapes=[
            pltpu.VMEM((npad, hpad), jnp.bfloat16),
        ],
        compiler_params=pltpu.CompilerParams(
            dimension_semantics=("arbitrary", "arbitrary"),
            vmem_limit_bytes=_VMEM_LIMIT,
        ),
    )(adj_p, z1, b1_p, w2_p, b2_p, wfc_p, bfc_p)

    return out[:n, :nclass]


def kernel(adj_p, x, w1, b1, w2, b2, wfc, bfc):
    return _forward(adj_p, x, w1, b1, w2, b2, wfc, bfc)


# speedup vs baseline: 1.8008x; 1.0649x over previous
import functools

import jax
import jax.numpy as jnp
from jax.experimental import pallas as pl
from jax.experimental.pallas import tpu as pltpu

_LANE = 128
_VMEM_LIMIT = 48 * 1024 * 1024
_TM = 1024
_TK = 2048
_TZ = 1024


def _round_up(n, m):
    return ((n + m - 1) // m) * m


def _tile(n, cap):
    t = min(cap, n)
    while t > _LANE:
        if n % t == 0:
            return t
        t -= _LANE
    return _LANE


def _pad2(a, rows, cols):
    r, c = a.shape
    if (r, c) == (rows, cols):
        return a
    return jnp.pad(a, ((0, rows - r), (0, cols - c)))


def _z1_kernel(x_ref, w1_ref, z1_ref):
    xb = x_ref[...].astype(jnp.bfloat16)
    z1_ref[...] = jnp.dot(xb, w1_ref[...],
                          preferred_element_type=jnp.float32).astype(jnp.bfloat16)


def _agg_kernel(adj_ref, z1_ref, b1_ref, w2_ref, b2_ref, wfc_ref, bfc_ref,
                out_ref, z2_ref, *, tm):
    p = pl.program_id(0)
    i = pl.program_id(1)

    @pl.when(p == 0)
    def _():
        acc = jnp.dot(adj_ref[...], z1_ref[...],
                      preferred_element_type=jnp.float32)
        h = jnp.maximum(acc + b1_ref[...], 0.0)
        z2_ref[pl.ds(pl.multiple_of(i * tm, _LANE), tm), :] = jnp.dot(
            h.astype(jnp.bfloat16), w2_ref[...],
            preferred_element_type=jnp.float32).astype(jnp.bfloat16)

    @pl.when(p == 1)
    def _():
        acc = jnp.dot(adj_ref[...], z2_ref[...],
                      preferred_element_type=jnp.float32)
        h = acc + b2_ref[...]
        out_ref[...] = jnp.dot(h.astype(jnp.bfloat16), wfc_ref[...],
                               preferred_element_type=jnp.float32) + bfc_ref[...]


@jax.jit
def _forward(adj_p, x, w1, b1, w2, b2, wfc, bfc):
    n, nfeat = x.shape
    npad = adj_p.shape[0]
    nhid = w1.shape[1]
    nclass = wfc.shape[1]

    fpad = _round_up(nfeat, _LANE)
    hpad = _round_up(nhid, _LANE)
    cpad = _round_up(nclass, _LANE)

    x_p = _pad2(x, npad, fpad)
    w1_p = _pad2(w1, fpad, hpad).astype(jnp.bfloat16)
    b1_p = _pad2(b1, 1, hpad).astype(jnp.float32)
    w2_p = _pad2(w2, hpad, hpad).astype(jnp.bfloat16)
    b2_p = _pad2(b2, 1, hpad).astype(jnp.float32)
    wfc_p = _pad2(wfc, hpad, cpad).astype(jnp.bfloat16)
    bfc_p = _pad2(bfc, 1, cpad).astype(jnp.float32)

    tz = _tile(npad, _TZ)
    tm = _tile(npad, _TM)

    z1 = pl.pallas_call(
        _z1_kernel,
        out_shape=jax.ShapeDtypeStruct((npad, hpad), jnp.bfloat16),
        grid=(npad // tz,),
        in_specs=[
            pl.BlockSpec((tz, fpad), lambda i: (i, 0)),
            pl.BlockSpec((fpad, hpad), lambda i: (0, 0)),
        ],
        out_specs=pl.BlockSpec((tz, hpad), lambda i: (i, 0)),
        compiler_params=pltpu.CompilerParams(
            dimension_semantics=("arbitrary",),
            vmem_limit_bytes=_VMEM_LIMIT,
        ),
    )(x_p, w1_p)

    body = functools.partial(_agg_kernel, tm=tm)
    out = pl.pallas_call(
        body,
        out_shape=jax.ShapeDtypeStruct((npad, cpad), jnp.float32),
        grid=(2, npad // tm),
        in_specs=[
            pl.BlockSpec((tm, npad), lambda p, i: (i, 0)),
            pl.BlockSpec((npad, hpad), lambda p, i: (0, 0)),
            pl.BlockSpec((1, hpad), lambda p, i: (0, 0)),
            pl.BlockSpec((hpad, hpad), lambda p, i: (0, 0)),
            pl.BlockSpec((1, hpad), lambda p, i: (0, 0)),
            pl.BlockSpec((hpad, cpad), lambda p, i: (0, 0)),
            pl.BlockSpec((1, cpad), lambda p, i: (0, 0)),
        ],
        out_specs=pl.BlockSpec((tm, cpad), lambda p, i: (p * i, 0)),
        scratch_shapes=[
            pltpu.VMEM((npad, hpad), jnp.bfloat16),
        ],
        compiler_params=pltpu.CompilerParams(
            dimension_semantics=("arbitrary", "arbitrary"),
            vmem_limit_bytes=_VMEM_LIMIT,
        ),
    )(adj_p, z1, b1_p, w2_p, b2_p, wfc_p, bfc_p)

    return out[:n, :nclass]


def kernel(adj_p, x, w1, b1, w2, b2, wfc, bfc):
    return _forward(adj_p, x, w1, b1, w2, b2, wfc, bfc)
